# baseline (speedup 1.0000x reference)
import functools

import jax
import jax.numpy as jnp
from jax import lax
from jax.experimental import pallas as pl
from jax.experimental.pallas import tpu as pltpu
from jax.experimental.pallas import tpu_sc as plsc

VOCAB = 100000
EMB = 2
BATCH = 4096

BB = 256
BV = 2048


def _gather_sc(xi, table_flat):
    info = plsc.get_sparse_core_info()
    nw = info.num_cores * info.num_subcores
    bpw = BATCH // nw
    mesh = plsc.VectorSubcoreMesh(core_axis_name="c", subcore_axis_name="s")

    @functools.partial(
        pl.kernel,
        mesh=mesh,
        compiler_params=pltpu.CompilerParams(use_tc_tiling_on_sc=False),
        out_type=jax.ShapeDtypeStruct((EMB, BATCH), jnp.float32),
        scratch_types=[
            pltpu.VMEM((bpw,), jnp.int32),
            pltpu.VMEM((bpw,), jnp.int32),
            pltpu.VMEM((bpw,), jnp.int32),
            pltpu.VMEM((bpw,), jnp.float32),
            pltpu.VMEM((bpw,), jnp.float32),
            pltpu.SemaphoreType.DMA,
            pltpu.SemaphoreType.DMA,
        ],
    )
    def sc_kernel(x_hbm, table_hbm, out_hbm,
                  idx_v, idx0_v, idx1_v, e0_v, e1_v, sem0, sem1):
        wid = lax.axis_index("s") * info.num_cores + lax.axis_index("c")
        base = wid * bpw
        pltpu.sync_copy(x_hbm.at[pl.ds(base, bpw)], idx_v)
        for k in range(bpw // 16):
            v = idx_v[pl.ds(k * 16, 16)]
            idx0_v[pl.ds(k * 16, 16)] = v * 2
            idx1_v[pl.ds(k * 16, 16)] = v * 2 + 1
        d0 = pltpu.async_copy(table_hbm.at[idx0_v], e0_v, sem0)
        d1 = pltpu.async_copy(table_hbm.at[idx1_v], e1_v, sem1)
        d0.wait()
        d1.wait()
        pltpu.sync_copy(e0_v, out_hbm.at[0, pl.ds(base, bpw)])
        pltpu.sync_copy(e1_v, out_hbm.at[1, pl.ds(base, bpw)])

    return sc_kernel(xi, table_flat)


def _dense_body(e_ref, w_ref, b_ref, out_ref):
    out_ref[...] = lax.dot_general(
        e_ref[...], w_ref[...],
        ((( 1,), (1,)), ((), ())),
        preferred_element_type=jnp.float32,
    ) + b_ref[...]


def _dense(e, W, b2):
    grid = (BATCH // BB, pl.cdiv(VOCAB, BV))
    return pl.pallas_call(
        _dense_body,
        grid=grid,
        in_specs=[
            pl.BlockSpec((BB, EMB), lambda i, j: (i, 0)),
            pl.BlockSpec((BV, EMB), lambda i, j: (j, 0)),
            pl.BlockSpec((1, BV), lambda i, j: (0, j)),
        ],
        out_specs=pl.BlockSpec((BB, BV), lambda i, j: (i, j)),
        out_shape=jax.ShapeDtypeStruct((BATCH, VOCAB), jnp.float32),
    )(e, W, b2)


def kernel(x, emb_table, W, b):
    xi = x.astype(jnp.int32)
    e01 = _gather_sc(xi, emb_table.reshape(VOCAB * EMB))
    e = e01.T
    b2 = b.reshape(1, VOCAB)
    logits = _dense(e, W, b2)
    return (logits, e)


# speedup vs baseline: 1.0157x; 1.0157x over previous
import functools

import jax
import jax.numpy as jnp
from jax import lax
from jax.experimental import pallas as pl
from jax.experimental.pallas import tpu as pltpu
from jax.experimental.pallas import tpu_sc as plsc

VOCAB = 100000
EMB = 2
BATCH = 4096

BB = 256
BV = 2048


def _gather_sc(xi, table_flat):
    info = plsc.get_sparse_core_info()
    nw = info.num_cores * info.num_subcores
    bpw = BATCH // nw
    mesh = plsc.VectorSubcoreMesh(core_axis_name="c", subcore_axis_name="s")

    @functools.partial(
        pl.kernel,
        mesh=mesh,
        compiler_params=pltpu.CompilerParams(use_tc_tiling_on_sc=False),
        out_type=jax.ShapeDtypeStruct((EMB, BATCH), jnp.float32),
        scratch_types=[
            pltpu.VMEM((bpw,), jnp.int32),
            pltpu.VMEM((bpw,), jnp.int32),
            pltpu.VMEM((bpw,), jnp.int32),
            pltpu.VMEM((bpw,), jnp.float32),
            pltpu.VMEM((bpw,), jnp.float32),
            pltpu.SemaphoreType.DMA,
            pltpu.SemaphoreType.DMA,
        ],
    )
    def sc_kernel(x_hbm, table_hbm, out_hbm,
                  idx_v, idx0_v, idx1_v, e0_v, e1_v, sem0, sem1):
        wid = lax.axis_index("s") * info.num_cores + lax.axis_index("c")
        base = wid * bpw
        pltpu.sync_copy(x_hbm.at[pl.ds(base, bpw)], idx_v)
        for k in range(bpw // 16):
            v = idx_v[pl.ds(k * 16, 16)]
            idx0_v[pl.ds(k * 16, 16)] = v * 2
            idx1_v[pl.ds(k * 16, 16)] = v * 2 + 1
        d0 = pltpu.async_copy(table_hbm.at[idx0_v], e0_v, sem0)
        d1 = pltpu.async_copy(table_hbm.at[idx1_v], e1_v, sem1)
        d0.wait()
        d1.wait()
        pltpu.sync_copy(e0_v, out_hbm.at[0, pl.ds(base, bpw)])
        pltpu.sync_copy(e1_v, out_hbm.at[1, pl.ds(base, bpw)])

    return sc_kernel(xi, table_flat)


def _dense_body(e_ref, w_ref, b_ref, out_ref):
    out_ref[...] = lax.dot_general(
        e_ref[...], w_ref[...],
        ((( 1,), (1,)), ((), ())),
        preferred_element_type=jnp.float32,
    ) + b_ref[...]


def _dense(e, W, b2):
    grid = (BATCH // BB, pl.cdiv(VOCAB, BV))
    return pl.pallas_call(
        _dense_body,
        grid=grid,
        in_specs=[
            pl.BlockSpec((BB, EMB), lambda i, j: (i, 0)),
            pl.BlockSpec((BV, EMB), lambda i, j: (j, 0)),
            pl.BlockSpec((1, BV), lambda i, j: (0, j)),
        ],
        out_specs=pl.BlockSpec((BB, BV), lambda i, j: (i, j)),
        out_shape=jax.ShapeDtypeStruct((BATCH, VOCAB), jnp.float32),
    )(e, W, b2)


def kernel(x, emb_table, W, b):
    xi = x.astype(jnp.int32)
    e = jnp.take(emb_table, xi, axis=0)
    b2 = b.reshape(1, VOCAB)
    logits = _dense(e, W, b2)
    return (logits, e)


# speedup vs baseline: 1.2075x; 1.1888x over previous
import functools

import jax
import jax.numpy as jnp
from jax import lax
from jax.experimental import pallas as pl
from jax.experimental.pallas import tpu as pltpu
from jax.experimental.pallas import tpu_sc as plsc

VOCAB = 100000
EMB = 2
BATCH = 4096

BB = 512
BV = 4096


def _gather_sc(xi, table_flat):
    info = plsc.get_sparse_core_info()
    nw = info.num_cores * info.num_subcores
    bpw = BATCH // nw
    mesh = plsc.VectorSubcoreMesh(core_axis_name="c", subcore_axis_name="s")

    @functools.partial(
        pl.kernel,
        mesh=mesh,
        compiler_params=pltpu.CompilerParams(use_tc_tiling_on_sc=False),
        out_type=jax.ShapeDtypeStruct((EMB, BATCH), jnp.float32),
        scratch_types=[
            pltpu.VMEM((bpw,), jnp.int32),
            pltpu.VMEM((bpw,), jnp.int32),
            pltpu.VMEM((bpw,), jnp.int32),
            pltpu.VMEM((bpw,), jnp.float32),
            pltpu.VMEM((bpw,), jnp.float32),
            pltpu.SemaphoreType.DMA,
            pltpu.SemaphoreType.DMA,
        ],
    )
    def sc_kernel(x_hbm, table_hbm, out_hbm,
                  idx_v, idx0_v, idx1_v, e0_v, e1_v, sem0, sem1):
        wid = lax.axis_index("s") * info.num_cores + lax.axis_index("c")
        base = wid * bpw
        pltpu.sync_copy(x_hbm.at[pl.ds(base, bpw)], idx_v)
        for k in range(bpw // 16):
            v = idx_v[pl.ds(k * 16, 16)]
            idx0_v[pl.ds(k * 16, 16)] = v * 2
            idx1_v[pl.ds(k * 16, 16)] = v * 2 + 1
        d0 = pltpu.async_copy(table_hbm.at[idx0_v], e0_v, sem0)
        d1 = pltpu.async_copy(table_hbm.at[idx1_v], e1_v, sem1)
        d0.wait()
        d1.wait()
        pltpu.sync_copy(e0_v, out_hbm.at[0, pl.ds(base, bpw)])
        pltpu.sync_copy(e1_v, out_hbm.at[1, pl.ds(base, bpw)])

    return sc_kernel(xi, table_flat)


def _dense_body(e_ref, w_ref, b_ref, out_ref):
    out_ref[...] = lax.dot_general(
        e_ref[...], w_ref[...],
        ((( 1,), (1,)), ((), ())),
        preferred_element_type=jnp.float32,
    ) + b_ref[...]


def _dense(e, W, b2):
    grid = (BATCH // BB, pl.cdiv(VOCAB, BV))
    return pl.pallas_call(
        _dense_body,
        grid=grid,
        in_specs=[
            pl.BlockSpec((BB, EMB), lambda i, j: (i, 0)),
            pl.BlockSpec((BV, EMB), lambda i, j: (j, 0)),
            pl.BlockSpec((1, BV), lambda i, j: (0, j)),
        ],
        out_specs=pl.BlockSpec((BB, BV), lambda i, j: (i, j)),
        out_shape=jax.ShapeDtypeStruct((BATCH, VOCAB), jnp.float32),
    )(e, W, b2)


def kernel(x, emb_table, W, b):
    xi = x.astype(jnp.int32)
    e = jnp.take(emb_table, xi, axis=0)
    b2 = b.reshape(1, VOCAB)
    logits = _dense(e, W, b2)
    return (logits, e)


# speedup vs baseline: 1.3179x; 1.0914x over previous
import functools

import jax
import jax.numpy as jnp
from jax import lax
from jax.experimental import pallas as pl
from jax.experimental.pallas import tpu as pltpu
from jax.experimental.pallas import tpu_sc as plsc

VOCAB = 100000
EMB = 2
BATCH = 4096

BB = 32
BV = VOCAB


def _gather_sc(xi, table_flat):
    info = plsc.get_sparse_core_info()
    nw = info.num_cores * info.num_subcores
    bpw = BATCH // nw
    mesh = plsc.VectorSubcoreMesh(core_axis_name="c", subcore_axis_name="s")

    @functools.partial(
        pl.kernel,
        mesh=mesh,
        compiler_params=pltpu.CompilerParams(use_tc_tiling_on_sc=False),
        out_type=jax.ShapeDtypeStruct((EMB, BATCH), jnp.float32),
        scratch_types=[
            pltpu.VMEM((bpw,), jnp.int32),
            pltpu.VMEM((bpw,), jnp.int32),
            pltpu.VMEM((bpw,), jnp.int32),
            pltpu.VMEM((bpw,), jnp.float32),
            pltpu.VMEM((bpw,), jnp.float32),
            pltpu.SemaphoreType.DMA,
            pltpu.SemaphoreType.DMA,
        ],
    )
    def sc_kernel(x_hbm, table_hbm, out_hbm,
                  idx_v, idx0_v, idx1_v, e0_v, e1_v, sem0, sem1):
        wid = lax.axis_index("s") * info.num_cores + lax.axis_index("c")
        base = wid * bpw
        pltpu.sync_copy(x_hbm.at[pl.ds(base, bpw)], idx_v)
        for k in range(bpw // 16):
            v = idx_v[pl.ds(k * 16, 16)]
            idx0_v[pl.ds(k * 16, 16)] = v * 2
            idx1_v[pl.ds(k * 16, 16)] = v * 2 + 1
        d0 = pltpu.async_copy(table_hbm.at[idx0_v], e0_v, sem0)
        d1 = pltpu.async_copy(table_hbm.at[idx1_v], e1_v, sem1)
        d0.wait()
        d1.wait()
        pltpu.sync_copy(e0_v, out_hbm.at[0, pl.ds(base, bpw)])
        pltpu.sync_copy(e1_v, out_hbm.at[1, pl.ds(base, bpw)])

    return sc_kernel(xi, table_flat)


def _dense_body(e_ref, wt_ref, b_ref, out_ref):
    out_ref[...] = lax.dot_general(
        e_ref[...], wt_ref[...],
        (((1,), (0,)), ((), ())),
        preferred_element_type=jnp.float32,
    ) + b_ref[...]


def _dense(e, W, b2):
    grid = (BATCH // BB,)
    return pl.pallas_call(
        _dense_body,
        grid=grid,
        in_specs=[
            pl.BlockSpec((BB, EMB), lambda i: (i, 0)),
            pl.BlockSpec((EMB, BV), lambda i: (0, 0)),
            pl.BlockSpec((1, BV), lambda i: (0, 0)),
        ],
        out_specs=pl.BlockSpec((BB, BV), lambda i: (i, 0)),
        out_shape=jax.ShapeDtypeStruct((BATCH, VOCAB), jnp.float32),
    )(e, W, b2)


def kernel(x, emb_table, W, b):
    xi = x.astype(jnp.int32)
    e = jnp.take(emb_table, xi, axis=0)
    b2 = b.reshape(1, VOCAB)
    logits = _dense(e, W.T, b2)
    return (logits, e)
